# baseline (speedup 1.0000x reference)
import functools
import jax
import jax.numpy as jnp
from jax import lax
from jax.experimental import pallas as pl
from jax.experimental.pallas import tpu as pltpu
from jax.experimental.pallas import tpu_sc as plsc

_NOISE_SCALE = 0.1
_BT = 1024
_L = 16


def _score_body(noise_ref, x_ref, wg_ref, bg_ref, s_ref):
    s = jnp.dot(x_ref[...], wg_ref[...], preferred_element_type=jnp.float32)
    s_ref[...] = s + bg_ref[...] + noise_ref[...]


def _make_sc_gate(t_tokens, n_e):
    n_workers = 32
    chunk = t_tokens // n_workers
    n_groups = chunk // _L
    mesh = plsc.VectorSubcoreMesh(core_axis_name="c", subcore_axis_name="s")

    @functools.partial(
        pl.kernel,
        mesh=mesh,
        out_type=jax.ShapeDtypeStruct((n_e, t_tokens), jnp.float32),
        scratch_types=[
            pltpu.VMEM((n_e, chunk), jnp.float32),
            pltpu.VMEM((n_e, chunk), jnp.float32),
        ],
    )
    def sc_gate(s_hbm, c_hbm, s_v, c_v):
        wid = lax.axis_index("s") * 2 + lax.axis_index("c")
        base = wid * chunk
        pltpu.sync_copy(s_hbm.at[:, pl.ds(base, chunk)], s_v)
        for g in range(n_groups):
            sl = pl.ds(g * _L, _L)
            w = [s_v[e, sl] for e in range(n_e)]
            m = w[0]
            for e in range(1, n_e):
                m = jnp.maximum(m, w[e])
            p = [jnp.exp(we - m) for we in w]
            den = p[0]
            for e in range(1, n_e):
                den = den + p[e]
            w = [pe / den for pe in p]
            m1 = jnp.maximum(w[0], w[1])
            m2 = jnp.minimum(w[0], w[1])
            for e in range(2, n_e):
                gt1 = w[e] > m1
                m2 = jnp.where(gt1, m1, jnp.maximum(m2, w[e]))
                m1 = jnp.where(gt1, w[e], m1)
            taken1 = m1 - m1
            taken2 = m1 - m1
            for e in range(n_e):
                eq1 = jnp.where(w[e] == m1, 1.0, 0.0)
                eq2 = jnp.where(w[e] == m2, 1.0, 0.0)
                f1 = eq1 * (1.0 - taken1)
                f2 = eq2 * (1.0 - f1) * (1.0 - taken2)
                taken1 = taken1 + f1
                taken2 = taken2 + f2
                c_v[e, sl] = w[e] * (f1 + f2 - f1 * f2)
        pltpu.sync_copy(c_v, c_hbm.at[:, pl.ds(base, chunk)])

    return sc_gate


def _expert_body(c_ref, x_ref, wcat_ref, be_ref, out_ref, xcat_ref):
    n_e = c_ref.shape[1]
    dim = x_ref.shape[1]
    c = c_ref[...]
    xb = x_ref[...].astype(jnp.bfloat16)
    cb = c.astype(jnp.bfloat16)
    for e in range(n_e):
        xcat_ref[:, e * dim:(e + 1) * dim] = xb * cb[:, e:e + 1]
    out_ref[...] = (
        jnp.dot(xcat_ref[...], wcat_ref[...], preferred_element_type=jnp.float32)
        + jnp.dot(c, be_ref[...], preferred_element_type=jnp.float32))


@jax.jit
def _run(x, Wg, bg, We, be):
    orig_shape = x.shape
    dim = x.shape[-1]
    xf = x.reshape(-1, dim)
    t = xf.shape[0]
    n_e = Wg.shape[-1]
    f = We.shape[-1]
    noise = jax.random.normal(jax.random.key(42), (t, n_e), jnp.float32) * _NOISE_SCALE
    wcat = We.astype(jnp.bfloat16).reshape(n_e * dim, f)

    s_t = pl.pallas_call(
        _score_body,
        grid=(t // _BT,),
        in_specs=[
            pl.BlockSpec((_BT, n_e), lambda i: (i, 0)),
            pl.BlockSpec((_BT, dim), lambda i: (i, 0)),
            pl.BlockSpec((dim, n_e), lambda i: (0, 0)),
            pl.BlockSpec((1, n_e), lambda i: (0, 0)),
        ],
        out_specs=pl.BlockSpec((_BT, n_e), lambda i: (i, 0)),
        out_shape=jax.ShapeDtypeStruct((t, n_e), jnp.float32),
        compiler_params=pltpu.CompilerParams(
            dimension_semantics=("parallel",),
        ),
    )(noise, xf, Wg, bg.reshape(1, n_e))

    c = _make_sc_gate(t, n_e)(s_t.T).T

    out = pl.pallas_call(
        _expert_body,
        grid=(t // _BT,),
        in_specs=[
            pl.BlockSpec((_BT, n_e), lambda i: (i, 0)),
            pl.BlockSpec((_BT, dim), lambda i: (i, 0)),
            pl.BlockSpec((n_e * dim, f), lambda i: (0, 0)),
            pl.BlockSpec((n_e, f), lambda i: (0, 0)),
        ],
        out_specs=pl.BlockSpec((_BT, f), lambda i: (i, 0)),
        out_shape=jax.ShapeDtypeStruct((t, f), jnp.float32),
        scratch_shapes=[
            pltpu.VMEM((_BT, n_e * dim), jnp.bfloat16),
        ],
        compiler_params=pltpu.CompilerParams(
            dimension_semantics=("parallel",),
        ),
    )(c, xf, wcat, be)
    return out.reshape(orig_shape)


def kernel(x, Wg, bg, We, be):
    return _run(x, Wg, bg, We, be)


# speedup vs baseline: 1.2142x; 1.2142x over previous
import jax
import jax.numpy as jnp
from jax.experimental import pallas as pl
from jax.experimental.pallas import tpu as pltpu

_NOISE_SCALE = 0.1
_BT = 1024


def _moe_body(noise_ref, x_ref, wg_ref, bg_ref, wcat_ref, be_ref, out_ref,
              xcat_ref):
    n_e = noise_ref.shape[1]
    dim = x_ref.shape[1]

    x = x_ref[...]
    s = jnp.dot(x, wg_ref[...], preferred_element_type=jnp.float32)
    s = s + bg_ref[...] + noise_ref[...]
    m = jnp.max(s, axis=-1, keepdims=True)
    p = jnp.exp(s - m)
    w = p / jnp.sum(p, axis=-1, keepdims=True)
    idx = jax.lax.broadcasted_iota(jnp.int32, w.shape, 1)
    m1 = jnp.max(w, axis=-1, keepdims=True)
    i1 = jnp.min(jnp.where(w == m1, idx, n_e), axis=-1, keepdims=True)
    wm = jnp.where(idx == i1, -jnp.inf, w)
    m2 = jnp.max(wm, axis=-1, keepdims=True)
    i2 = jnp.min(jnp.where(wm == m2, idx, n_e), axis=-1, keepdims=True)
    c = jnp.where(idx == i1, m1, jnp.where(idx == i2, m2, 0.0))

    xb = x.astype(jnp.bfloat16)
    cb = c.astype(jnp.bfloat16)
    for e in range(n_e):
        xcat_ref[:, e * dim:(e + 1) * dim] = xb * cb[:, e:e + 1]

    half = n_e * dim // 2
    out_ref[...] = (
        jnp.dot(xcat_ref[:, :half], wcat_ref[:half, :],
                preferred_element_type=jnp.float32)
        + jnp.dot(xcat_ref[:, half:], wcat_ref[half:, :],
                  preferred_element_type=jnp.float32)
        + jnp.dot(c, be_ref[...], preferred_element_type=jnp.float32))


@jax.jit
def _run(x, Wg, bg, We, be):
    orig_shape = x.shape
    dim = x.shape[-1]
    xf = x.reshape(-1, dim)
    t = xf.shape[0]
    n_e = Wg.shape[-1]
    f = We.shape[-1]
    noise = jax.random.normal(jax.random.key(42), (t, n_e), jnp.float32) * _NOISE_SCALE
    wcat = We.astype(jnp.bfloat16).reshape(n_e * dim, f)

    out = pl.pallas_call(
        _moe_body,
        grid=(t // _BT,),
        in_specs=[
            pl.BlockSpec((_BT, n_e), lambda i: (i, 0)),
            pl.BlockSpec((_BT, dim), lambda i: (i, 0)),
            pl.BlockSpec((dim, n_e), lambda i: (0, 0)),
            pl.BlockSpec((1, n_e), lambda i: (0, 0)),
            pl.BlockSpec((n_e * dim, f), lambda i: (0, 0)),
            pl.BlockSpec((n_e, f), lambda i: (0, 0)),
        ],
        out_specs=pl.BlockSpec((_BT, f), lambda i: (i, 0)),
        out_shape=jax.ShapeDtypeStruct((t, f), jnp.float32),
        scratch_shapes=[
            pltpu.VMEM((_BT, n_e * dim), jnp.bfloat16),
        ],
        compiler_params=pltpu.CompilerParams(
            dimension_semantics=("parallel",),
        ),
    )(noise, xf, Wg, bg.reshape(1, n_e), wcat, be)
    return out.reshape(orig_shape)


def kernel(x, Wg, bg, We, be):
    return _run(x, Wg, bg, We, be)
